# Initial kernel scaffold; baseline (speedup 1.0000x reference)
import jax
import jax.numpy as jnp
from jax.experimental import pallas as pl


def kernel(x, channel_params, offset_params):
    raise NotImplementedError("write your pallas kernel here")


# speedup vs baseline: 1.5944x; 1.5944x over previous
import functools

import jax
import jax.numpy as jnp
from jax import lax
from jax.experimental import pallas as pl
from jax.experimental.pallas import tpu as pltpu
from jax.experimental.pallas import tpu_sc as plsc

B, C, L = 16, 256, 4096
N = 128
W = 512
S = 528
NLANE = 16
NCHUNK = W // NLANE
NC, NS = 2, 16
NWORK = NC * NS
IPW = N // NWORK


def _sc_body(x_hbm, cp_hbm, op_hbm, out_hbm, cp_v, op_v, row0_v, row1_v,
             out_v, sem):
    wid = lax.axis_index("s") * NC + lax.axis_index("c")
    pltpu.sync_copy(cp_hbm, cp_v)
    pltpu.sync_copy(op_hbm, op_v)

    lanes = lax.iota(jnp.int32, NLANE)
    g = wid // 4
    sub = wid % 4

    cpv = cp_v[pl.ds(g * NLANE, NLANE)]
    opv = op_v[pl.ds(g * NLANE, NLANE)]

    dc = (C - 1) / (1.0 + jnp.exp(-cpv))
    c0v = jnp.minimum(dc.astype(jnp.int32), C - 1)
    c1v = jnp.minimum(c0v + 1, C - 1)
    wcv = dc - c0v.astype(jnp.float32)
    t0 = (L - W) / (1.0 + jnp.exp(-opv))
    f0v = jnp.minimum(t0.astype(jnp.int32), L - W - 1)
    wtv = t0 - f0v.astype(jnp.float32)
    fav = jnp.minimum((f0v >> 3) << 3, L - S)
    dv = f0v - fav

    w00v = (1.0 - wtv) * (1.0 - wcv)
    w01v = (1.0 - wtv) * wcv
    w10v = wtv * (1.0 - wcv)
    w11v = wtv * wcv

    for k in range(IPW):
        lane = sub * IPW + k
        m = lanes == lane
        zi = jnp.zeros((), jnp.int32)
        zf = jnp.zeros((), jnp.float32)
        c0 = jnp.sum(jnp.where(m, c0v, zi))
        c1 = jnp.sum(jnp.where(m, c1v, zi))
        fa = pl.multiple_of(jnp.sum(jnp.where(m, fav, zi)), 8)
        d = jnp.sum(jnp.where(m, dv, zi))
        w00 = jnp.sum(jnp.where(m, w00v, zf))
        w01 = jnp.sum(jnp.where(m, w01v, zf))
        w10 = jnp.sum(jnp.where(m, w10v, zf))
        w11 = jnp.sum(jnp.where(m, w11v, zf))
        i_out = g * NLANE + lane
        idx_a = d + lanes
        idx_b = idx_a + 1

        def b_body(b, _, c0=c0, c1=c1, fa=fa, w00=w00, w01=w01, w10=w10,
                   w11=w11, i_out=i_out, idx_a=idx_a, idx_b=idx_b):
            pltpu.sync_copy(x_hbm.at[b, c0, pl.ds(fa, S)], row0_v)
            pltpu.sync_copy(x_hbm.at[b, c1, pl.ds(fa, S)], row1_v)
            for j in range(NCHUNK):
                ia = idx_a + (j * NLANE)
                ib = idx_b + (j * NLANE)
                a0 = plsc.load_gather(row0_v, [ia])
                a1 = plsc.load_gather(row1_v, [ia])
                b0 = plsc.load_gather(row0_v, [ib])
                b1 = plsc.load_gather(row1_v, [ib])
                out_v[pl.ds(j * NLANE, NLANE)] = (
                    w00 * a0 + w01 * a1 + w10 * b0 + w11 * b1)
            pltpu.sync_copy(out_v, out_hbm.at[b, i_out])
            return _

        lax.fori_loop(0, B, b_body, 0)


@jax.jit
def kernel(x, channel_params, offset_params):
    mesh = plsc.VectorSubcoreMesh(core_axis_name="c", subcore_axis_name="s")
    run = pl.kernel(
        _sc_body,
        out_type=jax.ShapeDtypeStruct((B, N, W), jnp.float32),
        mesh=mesh,
        scratch_types=[
            pltpu.VMEM((N,), jnp.float32),
            pltpu.VMEM((N,), jnp.float32),
            pltpu.VMEM((S,), jnp.float32),
            pltpu.VMEM((S,), jnp.float32),
            pltpu.VMEM((W,), jnp.float32),
            pltpu.SemaphoreType.DMA,
        ],
        compiler_params=pltpu.CompilerParams(
            use_tc_tiling_on_sc=False, needs_layout_passes=False),
    )
    return run(x, channel_params, offset_params)
